# baseline (speedup 1.0000x reference)
import functools

import numpy as np
import jax
import jax.numpy as jnp
from jax import lax
from jax.experimental import pallas as pl
from jax.experimental.pallas import tpu as pltpu
from jax.experimental.pallas import tpu_sc as plsc

_VOCAB = 1000000
_D = 64
_B = 32
_S = 2048

_NC = 2
_NS = 16
_NW = _NC * _NS
_CHUNK = _S // _NW
_L = 16

_TBLK = 4096
_H = 123 * _TBLK


def _positional_encoding():
    pos = np.arange(_S, dtype=np.float64)[:, None]
    i = np.arange(0, _D, 2, dtype=np.float64)
    angle = pos / (10000.0 ** (2.0 * i / _D))
    pe = np.zeros((_S, _D), dtype=np.float64)
    pe[:, 0::2] = np.sin(angle)
    pe[:, 1::2] = np.cos(angle)
    perm = np.concatenate([np.arange(0, 32, 2), np.arange(1, 32, 2),
                           np.arange(32, 64, 2), np.arange(33, 64, 2)])
    return jnp.asarray(pe[:, perm], dtype=jnp.float32)


def _reformat_kernel(lo_ref, hi_ref, out_ref):
    stacked = jnp.concatenate([lo_ref[...], hi_ref[...]], axis=0)
    out_ref[...] = lax.convert_element_type(
        lax.transpose(stacked, (1, 0)), jnp.bfloat16)


def _reformat(table_t):
    half_blocks = _H // _TBLK
    return pl.pallas_call(
        _reformat_kernel,
        grid=(half_blocks,),
        in_specs=[
            pl.BlockSpec((_D, _TBLK), lambda i: (0, i)),
            pl.BlockSpec(
                (_D, _TBLK),
                lambda i: (0, jnp.minimum(i + half_blocks,
                                          (_VOCAB - 1) // _TBLK))),
        ],
        out_specs=pl.BlockSpec((_TBLK, 2 * _D), lambda i: (i, 0)),
        out_shape=jax.ShapeDtypeStruct((_H, 2 * _D), jnp.bfloat16),
    )(table_t, table_t)


def _build_sc_kernel():
    mesh = plsc.VectorSubcoreMesh(
        core_axis_name="c", subcore_axis_name="s",
        num_cores=_NC, num_subcores=_NS,
    )

    @functools.partial(
        pl.kernel,
        mesh=mesh,
        compiler_params=pltpu.CompilerParams(
            use_tc_tiling_on_sc=False, needs_layout_passes=False),
        out_type=jax.ShapeDtypeStruct((_B, _S, _D), jnp.float32),
        scratch_types=[
            pltpu.VMEM((_B, _CHUNK), jnp.int32),
            pltpu.VMEM((_CHUNK, _D), jnp.float32),
            pltpu.VMEM((2, _CHUNK, _D), jnp.bfloat16),
            pltpu.VMEM((_CHUNK, _D), jnp.float32),
            pltpu.SemaphoreType.DMA((2,)),
            pltpu.SemaphoreType.DMA,
        ],
    )
    def emb_kernel(x_h, table_h, pe_h, out_h, idx_v, pe_v, buf, obuf,
                   sems, isem):
        wid = lax.axis_index("s") * _NC + lax.axis_index("c")
        base = wid * _CHUNK

        pltpu.sync_copy(pe_h.at[pl.ds(base, _CHUNK), :], pe_v)
        idx_copies = [
            pltpu.async_copy(
                x_h.at[pl.ds(b * _S + base, _CHUNK)], idx_v.at[b], isem)
            for b in range(_B)
        ]
        for cp in idx_copies:
            cp.wait()

        def remap(i, _):
            b = i // (_CHUNK // _L)
            g = i % (_CHUNK // _L)
            sl = pl.ds(g * _L, _L)
            t = idx_v[b, sl]
            h = jnp.where(t >= jnp.int32(_H),
                          jnp.int32(1), jnp.int32(0))
            idx_v[b, sl] = t * 2 - h * jnp.int32(2 * _H - 1)
            return 0
        lax.fori_loop(0, _B * (_CHUNK // _L), remap, 0)

        copies = [None, None]
        copies[0] = pltpu.async_copy(
            table_h.at[idx_v.at[0]], buf.at[0], sems.at[0])

        for b in range(_B):
            slot = b % 2
            copies[slot].wait()
            if b + 1 < _B:
                nxt = (b + 1) % 2
                copies[nxt] = pltpu.async_copy(
                    table_h.at[idx_v.at[b + 1]], buf.at[nxt], sems.at[nxt])

            io2 = lax.iota(jnp.int32, _L) * 2

            def add_row(r, _):
                rvec = jnp.full((_L,), 0, jnp.int32) + r
                for g in range(2):
                    v = buf[slot, r, pl.ds(g * 32, 32)]
                    a, bb = plsc.unpack(
                        v, format=plsc.PackFormat.INTERLEAVED)
                    cols = io2 + g * 32
                    plsc.store_scatter(
                        obuf, [rvec, cols],
                        a + pe_v[r, pl.ds(g * 32, _L)])
                    plsc.store_scatter(
                        obuf, [rvec, cols + 1],
                        bb + pe_v[r, pl.ds(g * 32 + _L, _L)])
                return 0

            lax.fori_loop(0, _CHUNK, add_row, 0)
            pltpu.sync_copy(obuf, out_h.at[b, pl.ds(base, _CHUNK), :])

    return emb_kernel


_SC_KERNEL = None


def kernel(x, table):
    global _SC_KERNEL
    if _SC_KERNEL is None:
        _SC_KERNEL = _build_sc_kernel()
    t2 = _reformat(table.T)
    t_lin = jnp.reshape(t2, (2 * _H, _D))
    return _SC_KERNEL(jnp.reshape(x, (_B * _S,)), t_lin,
                      _positional_encoding())


# speedup vs baseline: 2.6279x; 2.6279x over previous
import functools

import numpy as np
import jax
import jax.numpy as jnp
from jax import lax
from jax.experimental import pallas as pl
from jax.experimental.pallas import tpu as pltpu
from jax.experimental.pallas import tpu_sc as plsc

_VOCAB = 1000000
_D = 64
_B = 32
_S = 2048

_NC = 2
_NS = 16
_NW = _NC * _NS
_CHUNK = _S // _NW
_L = 16

_TBLK = 8192
_H = 62 * _TBLK


def _positional_encoding():
    pos = np.arange(_S, dtype=np.float64)[:, None]
    i = np.arange(0, _D, 2, dtype=np.float64)
    angle = pos / (10000.0 ** (2.0 * i / _D))
    pe = np.zeros((_S, _D), dtype=np.float64)
    pe[:, 0::2] = np.sin(angle)
    pe[:, 1::2] = np.cos(angle)
    return jnp.asarray(pe, dtype=jnp.float32)


def _reformat_kernel(lo_ref, hi_ref, out_ref):
    stacked = jnp.concatenate([lo_ref[...], hi_ref[...]], axis=0)
    out_ref[...] = lax.transpose(stacked, (1, 0))


def _reformat(table_t):
    half_blocks = _H // _TBLK
    return pl.pallas_call(
        _reformat_kernel,
        grid=(half_blocks,),
        in_specs=[
            pl.BlockSpec((_D, _TBLK), lambda i: (0, i)),
            pl.BlockSpec(
                (_D, _TBLK),
                lambda i: (0, jnp.minimum(i + half_blocks,
                                          (_VOCAB - 1) // _TBLK))),
        ],
        out_specs=pl.BlockSpec((_TBLK, 2 * _D), lambda i: (i, 0)),
        out_shape=jax.ShapeDtypeStruct((_H, 2 * _D), jnp.float32),
    )(table_t, table_t)


def _build_sc_kernel():
    mesh = plsc.VectorSubcoreMesh(
        core_axis_name="c", subcore_axis_name="s",
        num_cores=_NC, num_subcores=_NS,
    )

    @functools.partial(
        pl.kernel,
        mesh=mesh,
        compiler_params=pltpu.CompilerParams(use_tc_tiling_on_sc=False),
        out_type=jax.ShapeDtypeStruct((_B, _S, _D), jnp.float32),
        scratch_types=[
            pltpu.VMEM((_B, _CHUNK), jnp.int32),
            pltpu.VMEM((_CHUNK, _D), jnp.float32),
            pltpu.VMEM((2, _CHUNK, _D), jnp.float32),
            pltpu.SemaphoreType.DMA((2,)),
            pltpu.SemaphoreType.DMA,
        ],
    )
    def emb_kernel(x_h, table_h, pe_h, out_h, idx_v, pe_v, buf, sems, isem):
        wid = lax.axis_index("s") * _NC + lax.axis_index("c")
        base = wid * _CHUNK

        pltpu.sync_copy(pe_h.at[pl.ds(base, _CHUNK), :], pe_v)
        idx_copies = [
            pltpu.async_copy(
                x_h.at[pl.ds(b * _S + base, _CHUNK)], idx_v.at[b], isem)
            for b in range(_B)
        ]
        for cp in idx_copies:
            cp.wait()

        def remap(i, _):
            b = i // (_CHUNK // _L)
            g = i % (_CHUNK // _L)
            sl = pl.ds(g * _L, _L)
            t = idx_v[b, sl]
            h = jnp.where(t >= jnp.int32(_H),
                          jnp.int32(1), jnp.int32(0))
            idx_v[b, sl] = t * 2 - h * jnp.int32(2 * _H - 1)
            return 0
        lax.fori_loop(0, _B * (_CHUNK // _L), remap, 0)

        copies = [None, None]
        copies[0] = pltpu.async_copy(
            table_h.at[idx_v.at[0]], buf.at[0], sems.at[0])

        for b in range(_B):
            slot = b % 2
            copies[slot].wait()
            if b + 1 < _B:
                nxt = (b + 1) % 2
                copies[nxt] = pltpu.async_copy(
                    table_h.at[idx_v.at[b + 1]], buf.at[nxt], sems.at[nxt])

            def add_row(r, _):
                for j in range(_D // _L):
                    sl = pl.ds(j * _L, _L)
                    buf[slot, r, sl] = buf[slot, r, sl] + pe_v[r, sl]
                return 0

            lax.fori_loop(0, _CHUNK, add_row, 0)
            pltpu.sync_copy(buf.at[slot], out_h.at[b, pl.ds(base, _CHUNK), :])

    return emb_kernel


_SC_KERNEL = None


def kernel(x, table):
    global _SC_KERNEL
    if _SC_KERNEL is None:
        _SC_KERNEL = _build_sc_kernel()
    t2 = _reformat(table.T)
    t_lin = jnp.reshape(t2, (2 * _H, _D))
    return _SC_KERNEL(jnp.reshape(x, (_B * _S,)), t_lin,
                      _positional_encoding())


# speedup vs baseline: 2.6911x; 1.0240x over previous
import functools

import numpy as np
import jax
import jax.numpy as jnp
from jax import lax
from jax.experimental import pallas as pl
from jax.experimental.pallas import tpu as pltpu
from jax.experimental.pallas import tpu_sc as plsc

_VOCAB = 1000000
_D = 64
_B = 32
_S = 2048

_NC = 2
_NS = 16
_NW = _NC * _NS
_CHUNK = _S // _NW
_L = 16

_TBLK = 16384
_H = 31 * _TBLK


def _positional_encoding():
    pos = np.arange(_S, dtype=np.float64)[:, None]
    i = np.arange(0, _D, 2, dtype=np.float64)
    angle = pos / (10000.0 ** (2.0 * i / _D))
    pe = np.zeros((_S, _D), dtype=np.float64)
    pe[:, 0::2] = np.sin(angle)
    pe[:, 1::2] = np.cos(angle)
    return jnp.asarray(pe, dtype=jnp.float32)


def _reformat_kernel(lo_ref, hi_ref, out_ref):
    stacked = jnp.concatenate([lo_ref[...], hi_ref[...]], axis=0)
    out_ref[...] = lax.transpose(stacked, (1, 0))


def _reformat(table_t):
    half_blocks = _H // _TBLK
    return pl.pallas_call(
        _reformat_kernel,
        grid=(half_blocks,),
        in_specs=[
            pl.BlockSpec((_D, _TBLK), lambda i: (0, i)),
            pl.BlockSpec(
                (_D, _TBLK),
                lambda i: (0, jnp.minimum(i + half_blocks,
                                          (_VOCAB - 1) // _TBLK))),
        ],
        out_specs=pl.BlockSpec((_TBLK, 2 * _D), lambda i: (i, 0)),
        out_shape=jax.ShapeDtypeStruct((_H, 2 * _D), jnp.float32),
    )(table_t, table_t)


def _build_sc_kernel():
    mesh = plsc.VectorSubcoreMesh(
        core_axis_name="c", subcore_axis_name="s",
        num_cores=_NC, num_subcores=_NS,
    )

    @functools.partial(
        pl.kernel,
        mesh=mesh,
        compiler_params=pltpu.CompilerParams(use_tc_tiling_on_sc=False),
        out_type=jax.ShapeDtypeStruct((_B, _S, _D), jnp.float32),
        scratch_types=[
            pltpu.VMEM((_B, _CHUNK), jnp.int32),
            pltpu.VMEM((_CHUNK, _D), jnp.float32),
            pltpu.VMEM((2, _CHUNK, _D), jnp.float32),
            pltpu.SemaphoreType.DMA((2,)),
            pltpu.SemaphoreType.DMA,
        ],
    )
    def emb_kernel(x_h, table_h, pe_h, out_h, idx_v, pe_v, buf, sems, isem):
        wid = lax.axis_index("s") * _NC + lax.axis_index("c")
        base = wid * _CHUNK

        pltpu.sync_copy(pe_h.at[pl.ds(base, _CHUNK), :], pe_v)
        idx_copies = [
            pltpu.async_copy(
                x_h.at[pl.ds(b * _S + base, _CHUNK)], idx_v.at[b], isem)
            for b in range(_B)
        ]
        for cp in idx_copies:
            cp.wait()

        def remap(i, _):
            b = i // (_CHUNK // _L)
            g = i % (_CHUNK // _L)
            sl = pl.ds(g * _L, _L)
            t = idx_v[b, sl]
            h = jnp.where(t >= jnp.int32(_H),
                          jnp.int32(1), jnp.int32(0))
            idx_v[b, sl] = t * 2 - h * jnp.int32(2 * _H - 1)
            return 0
        lax.fori_loop(0, _B * (_CHUNK // _L), remap, 0)

        copies = [None, None]
        copies[0] = pltpu.async_copy(
            table_h.at[idx_v.at[0]], buf.at[0], sems.at[0])

        for b in range(_B):
            slot = b % 2
            copies[slot].wait()
            if b + 1 < _B:
                nxt = (b + 1) % 2
                copies[nxt] = pltpu.async_copy(
                    table_h.at[idx_v.at[b + 1]], buf.at[nxt], sems.at[nxt])

            def add_row(r, _):
                for j in range(_D // _L):
                    sl = pl.ds(j * _L, _L)
                    buf[slot, r, sl] = buf[slot, r, sl] + pe_v[r, sl]
                return 0

            lax.fori_loop(0, _CHUNK, add_row, 0)
            pltpu.sync_copy(buf.at[slot], out_h.at[b, pl.ds(base, _CHUNK), :])

    return emb_kernel


_SC_KERNEL = None


def kernel(x, table):
    global _SC_KERNEL
    if _SC_KERNEL is None:
        _SC_KERNEL = _build_sc_kernel()
    t2 = _reformat(table.T)
    t_lin = jnp.reshape(t2, (2 * _H, _D))
    return _SC_KERNEL(jnp.reshape(x, (_B * _S,)), t_lin,
                      _positional_encoding())
